# baseline (speedup 1.0000x reference)
import functools

import jax
import jax.numpy as jnp
from jax import lax
from jax.experimental import pallas as pl
from jax.experimental.pallas import tpu as pltpu
from jax.experimental.pallas import tpu_sc as plsc

N_NODES = 10000
N_EDGES = 320000
C = 128

NUM_CORES = 2
NUM_SUBCORES = 16
NW = NUM_CORES * NUM_SUBCORES
CHUNK = 80
NCHUNK = 128
EDGES_PER_W = CHUNK * NCHUNK
E_PAD = NW * EDGES_PER_W
NPIECE = 8
PCHUNK = NCHUNK // NPIECE
N_PAD = 10240
ROWS_PER_SUB = N_PAD // NUM_SUBCORES
ZROWS = 64


def _agg_body(h_hbm, src_hbm, dst_hbm, acc_out, src_v, dst_v, rows_v,
              stage_v, acc_sh, gsem0, gsem1):
    cid = lax.axis_index("c")
    sid = lax.axis_index("s")
    wid = cid * NUM_SUBCORES + sid

    zero16 = jnp.zeros((16,), jnp.float32)

    def zrow(i, _):
        for l in range(C // 16):
            stage_v[i, pl.ds(l * 16, 16)] = zero16
        return 0

    lax.fori_loop(0, ZROWS, zrow, 0)

    def zcopy(r, _):
        base = sid * ROWS_PER_SUB + r * ZROWS
        pltpu.sync_copy(stage_v, acc_sh.at[pl.ds(base, ZROWS)])
        return 0

    lax.fori_loop(0, ROWS_PER_SUB // ZROWS, zcopy, 0)

    plsc.subcore_barrier()

    def gstart(j, b, sem):
        pltpu.async_copy(h_hbm.at[src_v.at[j]], rows_v.at[b], sem)

    def gwait(j, b, sem):
        pltpu.make_async_copy(h_hbm.at[src_v.at[j]], rows_v.at[b],
                              sem).wait()

    def piece_body(p, _):
        pltpu.sync_copy(src_hbm.at[wid, p], src_v)
        pltpu.sync_copy(dst_hbm.at[wid, p], dst_v)

        gstart(0, 0, gsem0)

        def pair_body(i, _):
            gstart(2 * i + 1, 1, gsem1)
            gwait(2 * i, 0, gsem0)
            pltpu.sync_copy(rows_v.at[0], acc_sh.at[dst_v.at[2 * i]],
                            add=True)

            @pl.when(i < PCHUNK // 2 - 1)
            def _():
                gstart(2 * i + 2, 0, gsem0)

            gwait(2 * i + 1, 1, gsem1)
            pltpu.sync_copy(rows_v.at[1], acc_sh.at[dst_v.at[2 * i + 1]],
                            add=True)
            return 0

        lax.fori_loop(0, PCHUNK // 2, pair_body, 0)
        return 0

    lax.fori_loop(0, NPIECE, piece_body, 0)

    plsc.subcore_barrier()

    def out_copy(r, _):
        base = sid * ROWS_PER_SUB + r * ZROWS
        pltpu.sync_copy(acc_sh.at[pl.ds(base, ZROWS)], stage_v)
        pltpu.sync_copy(stage_v, acc_out.at[cid, pl.ds(base, ZROWS)])
        return 0

    lax.fori_loop(0, ROWS_PER_SUB // ZROWS, out_copy, 0)


_agg = pl.kernel(
    _agg_body,
    out_type=[jax.ShapeDtypeStruct((NUM_CORES, N_PAD, C), jnp.float32)],
    mesh=plsc.VectorSubcoreMesh(core_axis_name="c", subcore_axis_name="s"),
    scratch_types=[
        pltpu.VMEM((PCHUNK, CHUNK), jnp.int32),
        pltpu.VMEM((PCHUNK, CHUNK), jnp.int32),
        pltpu.VMEM((2, CHUNK, C), jnp.float32),
        pltpu.VMEM((ZROWS, C), jnp.float32),
        pltpu.VMEM_SHARED((N_PAD, C), jnp.float32),
        pltpu.SemaphoreType.DMA,
        pltpu.SemaphoreType.DMA,
    ],
)


BR = 1000


def _tc_body(with_relu, acc_ref, degp_ref, h_ref, wl_ref, bl_ref, wr_ref,
             *outs):
    agg = acc_ref[0] + acc_ref[1]
    deg = degp_ref[0][:, 0:1] + degp_ref[1][:, 0:1]
    mean = agg * (1.0 / jnp.maximum(deg, 1.0))
    out = (jnp.dot(mean, wl_ref[...], preferred_element_type=jnp.float32)
           + jnp.dot(h_ref[...], wr_ref[...], preferred_element_type=jnp.float32)
           + bl_ref[...])
    outs[0][...] = out
    if with_relu:
        outs[1][...] = jnp.maximum(out, 0.0)


def _tc_layer(with_relu, acc, degp, h, wl, bl, wr):
    grid = (N_NODES // BR,)
    out_shape = [jax.ShapeDtypeStruct((N_NODES, C), jnp.float32)]
    out_specs = [pl.BlockSpec((BR, C), lambda i: (i, 0))]
    if with_relu:
        out_shape.append(jax.ShapeDtypeStruct((N_NODES, C), jnp.float32))
        out_specs.append(pl.BlockSpec((BR, C), lambda i: (i, 0)))
    return pl.pallas_call(
        functools.partial(_tc_body, with_relu),
        grid=grid,
        in_specs=[
            pl.BlockSpec((NUM_CORES, BR, C), lambda i: (0, i, 0)),
            pl.BlockSpec((NUM_CORES, BR, C), lambda i: (0, i, 0)),
            pl.BlockSpec((BR, C), lambda i: (i, 0)),
            pl.BlockSpec((C, C), lambda i: (0, 0)),
            pl.BlockSpec((1, C), lambda i: (0, 0)),
            pl.BlockSpec((C, C), lambda i: (0, 0)),
        ],
        out_specs=out_specs,
        out_shape=out_shape,
    )(acc, degp, h, wl, bl, wr)


def kernel(x, edge_index, Wl0, bl0, Wr0, Wl1, bl1, Wr1, Wl2, bl2, Wr2):
    pad = E_PAD - N_EDGES
    src = jnp.concatenate(
        [edge_index[0].astype(jnp.int32), jnp.zeros((pad,), jnp.int32)])
    dst = jnp.concatenate(
        [edge_index[1].astype(jnp.int32),
         jnp.full((pad,), N_NODES, jnp.int32)])
    src = src.reshape(NW, NPIECE, PCHUNK, CHUNK)
    dst = dst.reshape(NW, NPIECE, PCHUNK, CHUNK)

    ones_table = jnp.ones((8, C), jnp.float32)
    zeros_idx = jnp.zeros((NW, NPIECE, PCHUNK, CHUNK), jnp.int32)
    (degp,) = _agg(ones_table, zeros_idx, dst)

    (acc0,) = _agg(x, src, dst)
    out0, h1 = _tc_layer(True, acc0, degp, x, Wl0, bl0.reshape(1, C), Wr0)

    (acc1,) = _agg(h1, src, dst)
    out1, h2 = _tc_layer(True, acc1, degp, h1, Wl1, bl1.reshape(1, C), Wr1)

    (acc2,) = _agg(h2, src, dst)
    (out2,) = _tc_layer(False, acc2, degp, h2, Wl2, bl2.reshape(1, C), Wr2)

    return jnp.stack([out0, out1, out2], axis=1)
